# baseline (speedup 1.0000x reference)
import functools

import jax
import jax.numpy as jnp
from jax import lax
from jax.experimental import pallas as pl
from jax.experimental.pallas import tpu as pltpu, tpu_sc as plsc

N = 10000
E = 320000
G = 16
DN = 128
DE = 16
DG = 32
H = 128

NC = 2
NS = 16
NW = NC * NS
CK = 128
NFULL = 80
EPW = NFULL * CK
E2 = EPW * NW
TBL = 10176

NB = 200
NGRID = N // NB
EBLK = 512
EGRID = E2 // EBLK

_SC_MESH = plsc.VectorSubcoreMesh(
    core_axis_name="c", subcore_axis_name="s", num_cores=NC, num_subcores=NS)


def _add_rows(dst, src, nrows):
    def body(i, _):
        for j in range(8):
            sl = pl.ds(j * 16, 16)
            dst[i, sl] = dst[i, sl] + src[i, sl]
        return 0
    lax.fori_loop(0, nrows, body, 0)


def _p1_body(a_hbm, b_hbm, row_hbm, col_hbm, g_hbm,
             rowv0, colv0, av0, bv0, rowv1, colv1, av1, bv1, sema, semb):
    wid = lax.axis_index("s") * NC + lax.axis_index("c")
    base = pl.multiple_of(wid * EPW, 128)

    def idx_load(off, rv, cv):
        pltpu.sync_copy(row_hbm.at[pl.ds(off, CK)], rv)
        pltpu.sync_copy(col_hbm.at[pl.ds(off, CK)], cv)

    def gather(rv, cv, ab, bb):
        pltpu.async_copy(a_hbm.at[rv], ab, sema)
        pltpu.async_copy(b_hbm.at[cv], bb, semb)

    def finish(rv, cv, ab, bb, off):
        pltpu.make_async_copy(a_hbm.at[rv], ab, sema).wait()
        pltpu.make_async_copy(b_hbm.at[cv], bb, semb).wait()
        _add_rows(ab, bb, CK)
        pltpu.sync_copy(ab, g_hbm.at[pl.ds(off, CK)])

    idx_load(base, rowv0, colv0)
    gather(rowv0, colv0, av0, bv0)
    idx_load(base + CK, rowv1, colv1)

    def body(c, _):
        t0 = pl.multiple_of(base + 2 * c * CK, 128)
        gather(rowv1, colv1, av1, bv1)
        finish(rowv0, colv0, av0, bv0, t0)
        idx_load(pl.multiple_of(t0 + 2 * CK, 128), rowv0, colv0)
        gather(rowv0, colv0, av0, bv0)
        finish(rowv1, colv1, av1, bv1, pl.multiple_of(t0 + CK, 128))
        idx_load(pl.multiple_of(t0 + 3 * CK, 128), rowv1, colv1)
        return 0
    lax.fori_loop(0, NFULL // 2 - 1, body, 0)

    gather(rowv1, colv1, av1, bv1)
    finish(rowv0, colv0, av0, bv0,
           pl.multiple_of(base + (NFULL - 2) * CK, 128))
    finish(rowv1, colv1, av1, bv1,
           pl.multiple_of(base + (NFULL - 1) * CK, 128))


@functools.partial(
    pl.kernel,
    out_type=jax.ShapeDtypeStruct((E2, DN), jnp.float32),
    mesh=_SC_MESH,
    scratch_types=[
        pltpu.VMEM((CK,), jnp.int32), pltpu.VMEM((CK,), jnp.int32),
        pltpu.VMEM((CK, DN), jnp.float32), pltpu.VMEM((CK, DN), jnp.float32),
        pltpu.VMEM((CK,), jnp.int32), pltpu.VMEM((CK,), jnp.int32),
        pltpu.VMEM((CK, DN), jnp.float32), pltpu.VMEM((CK, DN), jnp.float32),
        pltpu.SemaphoreType.DMA, pltpu.SemaphoreType.DMA,
    ],
)
def _p1(a_hbm, b_hbm, row_hbm, col_hbm, g_hbm, *rest):
    _p1_body(a_hbm, b_hbm, row_hbm, col_hbm, g_hbm, *rest)


def _zero_shared(zb, sh, sid):
    rps = TBL // NS
    zbase = sid * rps
    nfull = rps // CK

    def body(k, _):
        pltpu.sync_copy(zb, sh.at[pl.ds(zbase + k * CK, CK)])
        return 0
    lax.fori_loop(0, nfull, body, 0)
    rem = rps - nfull * CK
    if rem:
        pltpu.sync_copy(zb.at[pl.ds(0, rem)],
                        sh.at[pl.ds(zbase + nfull * CK, rem)])


def _p3a_body(c_hbm, stc_hbm, row_hbm, col_hbm, acc_out,
              rowv0, colv0, cv0, sv0, acc_sh, semc):
    cid = lax.axis_index("c")
    sid = lax.axis_index("s")
    wid = sid * NC + cid
    base = pl.multiple_of(wid * EPW, 128)

    zero16 = jnp.zeros((16,), jnp.float32)

    def fill(i, _):
        for j in range(8):
            cv0[i, pl.ds(j * 16, 16)] = zero16
        return 0
    lax.fori_loop(0, CK, fill, 0)
    _zero_shared(cv0, acc_sh, sid)
    plsc.subcore_barrier()

    def body(c, _):
        off = pl.multiple_of(base + c * CK, 128)
        pltpu.sync_copy(row_hbm.at[pl.ds(off, CK)], rowv0)
        pltpu.sync_copy(col_hbm.at[pl.ds(off, CK)], colv0)
        pltpu.async_copy(c_hbm.at[rowv0], cv0, semc)
        pltpu.sync_copy(stc_hbm.at[pl.ds(off, CK)], sv0)
        pltpu.make_async_copy(c_hbm.at[rowv0], cv0, semc).wait()
        _add_rows(cv0, sv0, CK)
        pltpu.sync_copy(cv0, acc_sh.at[colv0], add=True)
        return 0
    lax.fori_loop(0, NFULL, body, 0)

    plsc.subcore_barrier()

    @pl.when(sid == 0)
    def _():
        pltpu.sync_copy(acc_sh, acc_out.at[cid])


@functools.partial(
    pl.kernel,
    out_type=jax.ShapeDtypeStruct((NC, TBL, DN), jnp.float32),
    mesh=_SC_MESH,
    scratch_types=[
        pltpu.VMEM((CK,), jnp.int32), pltpu.VMEM((CK,), jnp.int32),
        pltpu.VMEM((CK, DN), jnp.float32), pltpu.VMEM((CK, DN), jnp.float32),
        pltpu.VMEM_SHARED((TBL, DN), jnp.float32),
        pltpu.SemaphoreType.DMA,
    ],
)
def _p3a(c_hbm, stc_hbm, row_hbm, col_hbm, acc_out, *rest):
    _p3a_body(c_hbm, stc_hbm, row_hbm, col_hbm, acc_out, *rest)


def _p3b_body(enew_hbm, row_hbm, racc_out, rowv0, ev0, ew, racc_sh):
    cid = lax.axis_index("c")
    sid = lax.axis_index("s")
    wid = sid * NC + cid
    base = pl.multiple_of(wid * EPW, 128)

    zero16 = jnp.zeros((16,), jnp.float32)

    def fill(i, _):
        for j in range(8):
            ew[i, pl.ds(j * 16, 16)] = zero16
        return 0
    lax.fori_loop(0, CK, fill, 0)
    _zero_shared(ew, racc_sh, sid)
    plsc.subcore_barrier()

    def body(c, _):
        off = pl.multiple_of(base + c * CK, 128)
        pltpu.sync_copy(row_hbm.at[pl.ds(off, CK)], rowv0)
        pltpu.sync_copy(enew_hbm.at[pl.ds(off, CK)], ev0)

        def cp(i, _):
            ew[i, pl.ds(0, 16)] = ev0[i, pl.ds(0, 16)]
            ew[i, pl.ds(16, 16)] = ev0[i, pl.ds(16, 16)]
            return 0
        lax.fori_loop(0, CK, cp, 0)
        pltpu.sync_copy(ew, racc_sh.at[rowv0], add=True)
        return 0
    lax.fori_loop(0, NFULL, body, 0)

    plsc.subcore_barrier()

    @pl.when(sid == 0)
    def _():
        pltpu.sync_copy(racc_sh, racc_out.at[cid])


@functools.partial(
    pl.kernel,
    out_type=jax.ShapeDtypeStruct((NC, TBL, DN), jnp.float32),
    mesh=_SC_MESH,
    scratch_types=[
        pltpu.VMEM((CK,), jnp.int32), pltpu.VMEM((CK, 2 * DE), jnp.float32),
        pltpu.VMEM((CK, DN), jnp.float32),
        pltpu.VMEM_SHARED((TBL, DN), jnp.float32),
    ],
)
def _p3b(enew_hbm, row_hbm, racc_out, *rest):
    _p3b_body(enew_hbm, row_hbm, racc_out, *rest)


def _p0_body(x_ref, oh_ref, w_ref, u_ref, wu_ref, b_ref, out_ref):
    uw = u_ref[...] @ wu_ref[...]
    t = oh_ref[...] @ uw
    pad = jnp.zeros((NB, 2 * DN), jnp.float32)
    out_ref[...] = (x_ref[...] @ w_ref[...] + b_ref[...]
                    + jnp.concatenate([t, pad], axis=1))


def _p0(x, oh, w, u, wu, b):
    return pl.pallas_call(
        _p0_body,
        grid=(NGRID,),
        in_specs=[
            pl.BlockSpec((NB, DN), lambda i: (i, 0)),
            pl.BlockSpec((NB, G), lambda i: (i, 0)),
            pl.BlockSpec((DN, 3 * DN), lambda i: (0, 0)),
            pl.BlockSpec((G, DG), lambda i: (0, 0)),
            pl.BlockSpec((DG, DN), lambda i: (0, 0)),
            pl.BlockSpec((1, 3 * DN), lambda i: (0, 0)),
        ],
        out_specs=pl.BlockSpec((NB, 3 * DN), lambda i: (i, 0)),
        out_shape=jax.ShapeDtypeStruct((N, 3 * DN), jnp.float32),
        compiler_params=pltpu.CompilerParams(
            dimension_semantics=("arbitrary",)),
    )(x, oh, w, u, wu, b)


def _p2_body(g_ref, ea_ref, wea_ref, w2_ref, w1b_ref, eb2_ref,
             enew_ref, enew32_ref, stc_ref):
    h = jnp.maximum(g_ref[...] + ea_ref[...] @ wea_ref[...], 0.0)
    enew = h @ w2_ref[...] + eb2_ref[...]
    enew_ref[...] = enew
    enew32_ref[...] = jnp.concatenate(
        [enew, jnp.ones((EBLK, DE), jnp.float32)], axis=1)
    m2 = w2_ref[...] @ w1b_ref[...]
    c2 = eb2_ref[...] @ w1b_ref[...]
    stc_ref[...] = h @ m2 + c2


def _p2(g, edge_attr, wea, ew2, w1b, eb2):
    return pl.pallas_call(
        _p2_body,
        grid=(EGRID,),
        in_specs=[
            pl.BlockSpec((EBLK, DN), lambda i: (i, 0)),
            pl.BlockSpec((EBLK, DE), lambda i: (i, 0)),
            pl.BlockSpec((DE, DN), lambda i: (0, 0)),
            pl.BlockSpec((DN, DE), lambda i: (0, 0)),
            pl.BlockSpec((DE, DN), lambda i: (0, 0)),
            pl.BlockSpec((1, DE), lambda i: (0, 0)),
        ],
        out_specs=[
            pl.BlockSpec((EBLK, DE), lambda i: (i, 0)),
            pl.BlockSpec((EBLK, 2 * DE), lambda i: (i, 0)),
            pl.BlockSpec((EBLK, DN), lambda i: (i, 0)),
        ],
        out_shape=[
            jax.ShapeDtypeStruct((E2, DE), jnp.float32),
            jax.ShapeDtypeStruct((E2, 2 * DE), jnp.float32),
            jax.ShapeDtypeStruct((E2, DN), jnp.float32),
        ],
        compiler_params=pltpu.CompilerParams(
            dimension_semantics=("arbitrary",)),
    )(g, edge_attr, wea, ew2, w1b, eb2)


def _p4_body(a0, a1, r0, r1, x_ref, oh_ref, u_ref,
             n1b1_, n1w2_, n1b2_, w2a, w2b, w2c, n2b1_, n2w2_, n2b2_,
             gw1_, gb1_, gw2_, gb2_,
             xnew_ref, nsum_ref, gacc_ref, unew_ref):
    i = pl.program_id(0)
    acct = a0[...] + a1[...]
    h1 = jnp.maximum(acct + n1b1_[...], 0.0) @ n1w2_[...] + n1b2_[...]
    uw2 = u_ref[...] @ w2c[...]
    pre2 = (x_ref[...] @ w2a[...] + h1 @ w2b[...]
            + oh_ref[...] @ uw2 + n2b1_[...])
    xn = jnp.maximum(pre2, 0.0) @ n2w2_[...] + n2b2_[...]
    xnew_ref[...] = xn
    oh = oh_ref[...]
    tdims = (((0,), (0,)), ((), ()))
    nsum_c = lax.dot_general(oh, xn, tdims,
                             preferred_element_type=jnp.float32)
    vals = jnp.concatenate(
        [(r0[...] + r1[...])[:, :2 * DE],
         jnp.ones((NB, DE), jnp.float32),
         jnp.zeros((NB, DN - 3 * DE), jnp.float32)], axis=1)
    gacc_c = lax.dot_general(oh, vals, tdims,
                             preferred_element_type=jnp.float32)

    @pl.when(i == 0)
    def _():
        nsum_ref[...] = nsum_c
        gacc_ref[...] = gacc_c

    @pl.when(i > 0)
    def _():
        nsum_ref[...] += nsum_c
        gacc_ref[...] += gacc_c

    @pl.when(i == NGRID - 1)
    def _():
        nsum = nsum_ref[...]
        gacc = gacc_ref[...]
        esum = gacc[:, :DE]
        ecnt = gacc[:, DE:DE + 1]
        ncnt = gacc[:, 2 * DE:2 * DE + 1]
        node_info = nsum / jnp.maximum(ncnt, 1.0)
        edge_info = esum / jnp.maximum(ecnt, 1.0)
        g_in = jnp.concatenate([u_ref[...], node_info, edge_info], axis=1)
        unew_ref[...] = (jnp.maximum(g_in @ gw1_[...] + gb1_[...], 0.0)
                         @ gw2_[...] + gb2_[...])


def _p4(acc0, acc1, racc0, racc1, x, oh, u,
        n1b1, n1w2, n1b2, w2a, w2b, w2c, n2b1, n2w2, n2b2,
        gw1, gb1, gw2, gb2):
    def full(shape):
        return pl.BlockSpec(shape, lambda i: tuple(0 for _ in shape))
    return pl.pallas_call(
        _p4_body,
        grid=(NGRID,),
        in_specs=[
            pl.BlockSpec((NB, DN), lambda i: (i, 0)),
            pl.BlockSpec((NB, DN), lambda i: (i, 0)),
            pl.BlockSpec((NB, DN), lambda i: (i, 0)),
            pl.BlockSpec((NB, DN), lambda i: (i, 0)),
            pl.BlockSpec((NB, DN), lambda i: (i, 0)),
            pl.BlockSpec((NB, G), lambda i: (i, 0)),
            full((G, DG)),
            full((1, DN)), full((DN, DN)), full((1, DN)),
            full((DN, DN)), full((DN, DN)), full((DG, DN)),
            full((1, DN)), full((DN, DN)), full((1, DN)),
            full((DG + DN + DE, DN)), full((1, DN)),
            full((DN, DG)), full((1, DG)),
        ],
        out_specs=[
            pl.BlockSpec((NB, DN), lambda i: (i, 0)),
            pl.BlockSpec((G, DN), lambda i: (0, 0)),
            pl.BlockSpec((G, DN), lambda i: (0, 0)),
            pl.BlockSpec((G, DG), lambda i: (0, 0)),
        ],
        out_shape=[
            jax.ShapeDtypeStruct((N, DN), jnp.float32),
            jax.ShapeDtypeStruct((G, DN), jnp.float32),
            jax.ShapeDtypeStruct((G, DN), jnp.float32),
            jax.ShapeDtypeStruct((G, DG), jnp.float32),
        ],
        compiler_params=pltpu.CompilerParams(
            dimension_semantics=("arbitrary",)),
    )(acc0, acc1, racc0, racc1, x, oh, u,
      n1b1, n1w2, n1b2, w2a, w2b, w2c, n2b1, n2w2, n2b2,
      gw1, gb1, gw2, gb2)


def kernel(x, edge_index, edge_attr, u, batch,
           ew1, eb1, ew2, eb2,
           n1w1, n1b1, n1w2, n1b2,
           n2w1, n2b1, n2w2, n2b2,
           gw1, gb1, gw2, gb2):
    pad_e = E2 - E
    row = jnp.concatenate([edge_index[0],
                           jnp.full((pad_e,), N, edge_index.dtype)])
    col = jnp.concatenate([edge_index[1],
                           jnp.full((pad_e,), N, edge_index.dtype)])
    ea_p = jnp.pad(edge_attr, ((0, pad_e), (0, 0)))
    oh = (batch[:, None] == jnp.arange(G, dtype=batch.dtype)[None, :])
    oh = oh.astype(jnp.float32)

    w0 = jnp.concatenate([ew1[:DN], ew1[DN:2 * DN], n1w1[:DN]], axis=1)
    b0 = jnp.concatenate([eb1, jnp.zeros((2 * DN,), jnp.float32)])[None, :]
    abc = jnp.pad(_p0(x, oh, w0, u, ew1[2 * DN + DE:], b0),
                  ((0, TBL - N), (0, 0)))
    a_t = abc[:, :DN]
    b_t = abc[:, DN:2 * DN]
    c_t = abc[:, 2 * DN:]

    g = _p1(a_t, b_t, row, col)

    e_new, e_new32, s_tc = _p2(g, ea_p, ew1[2 * DN:2 * DN + DE], ew2,
                               n1w1[DN:DN + DE], eb2[None, :])

    acc_p = _p3a(c_t, s_tc, row, col)
    racc_p = _p3b(e_new32, row)

    x_new, _, _, u_new = _p4(
        acc_p[0], acc_p[1], racc_p[0], racc_p[1],
        x, oh, u,
        n1b1[None, :], n1w2, n1b2[None, :],
        n2w1[:DN], n2w1[DN:DN + H], n2w1[DN + H:],
        n2b1[None, :], n2w2, n2b2[None, :],
        gw1, gb1[None, :], gw2, gb2[None, :])

    return (x_new, e_new[:E], u_new)


# speedup vs baseline: 1.1205x; 1.1205x over previous
import functools

import jax
import jax.numpy as jnp
from jax import lax
from jax.experimental import pallas as pl
from jax.experimental.pallas import tpu as pltpu, tpu_sc as plsc

N = 10000
E = 320000
G = 16
DN = 128
DE = 16
DG = 32
H = 128

NC = 2
NS = 16
NW = NC * NS
CK = 128
NFULL = 79
EPW = NFULL * CK
E2 = EPW * NW
TBL = 10176

NB = 200
NGRID = N // NB
EBLK = 512
EGRID = E2 // EBLK

_SC_MESH = plsc.VectorSubcoreMesh(
    core_axis_name="c", subcore_axis_name="s", num_cores=NC, num_subcores=NS)


def _add_rows(dst, src, nrows):
    def body(i, _):
        for j in range(8):
            sl = pl.ds(j * 16, 16)
            dst[i, sl] = dst[i, sl] + src[i, sl]
        return 0
    lax.fori_loop(0, nrows, body, 0)


def _p1_body(a_hbm, b_hbm, row_hbm, col_hbm, g_hbm,
             rowv, colv, av, bv, sem0, sem1):
    wid = lax.axis_index("s") * NC + lax.axis_index("c")
    base = pl.multiple_of(wid * EPW, 128)

    def body(c, _):
        off = pl.multiple_of(base + c * CK, 128)
        pltpu.sync_copy(row_hbm.at[pl.ds(off, CK)], rowv)
        pltpu.sync_copy(col_hbm.at[pl.ds(off, CK)], colv)
        d0 = pltpu.async_copy(a_hbm.at[rowv], av, sem0)
        d1 = pltpu.async_copy(b_hbm.at[colv], bv, sem1)
        d0.wait()
        d1.wait()
        _add_rows(av, bv, CK)
        pltpu.sync_copy(av, g_hbm.at[pl.ds(off, CK)])
        return 0
    lax.fori_loop(0, NFULL, body, 0)


@functools.partial(
    pl.kernel,
    out_type=jax.ShapeDtypeStruct((E2, DN), jnp.float32),
    mesh=_SC_MESH,
    scratch_types=[
        pltpu.VMEM((CK,), jnp.int32), pltpu.VMEM((CK,), jnp.int32),
        pltpu.VMEM((CK, DN), jnp.float32), pltpu.VMEM((CK, DN), jnp.float32),
        pltpu.SemaphoreType.DMA, pltpu.SemaphoreType.DMA,
    ],
)
def _p1(a_hbm, b_hbm, row_hbm, col_hbm, g_hbm, *rest):
    _p1_body(a_hbm, b_hbm, row_hbm, col_hbm, g_hbm, *rest)


def _zero_shared(zb, sh, sid):
    rps = TBL // NS
    zbase = sid * rps
    nfull = rps // CK

    def body(k, _):
        pltpu.sync_copy(zb, sh.at[pl.ds(zbase + k * CK, CK)])
        return 0
    lax.fori_loop(0, nfull, body, 0)
    rem = rps - nfull * CK
    if rem:
        pltpu.sync_copy(zb.at[pl.ds(0, rem)],
                        sh.at[pl.ds(zbase + nfull * CK, rem)])


def _p3a_body(c_hbm, stc_hbm, row_hbm, col_hbm, acc_out,
              rowv0, colv0, cv0, sv0, acc_sh, semc):
    cid = lax.axis_index("c")
    sid = lax.axis_index("s")
    wid = sid * NC + cid
    base = pl.multiple_of(wid * EPW, 128)

    zero16 = jnp.zeros((16,), jnp.float32)

    def fill(i, _):
        for j in range(8):
            cv0[i, pl.ds(j * 16, 16)] = zero16
        return 0
    lax.fori_loop(0, CK, fill, 0)
    _zero_shared(cv0, acc_sh, sid)
    plsc.subcore_barrier()

    def body(c, _):
        off = pl.multiple_of(base + c * CK, 128)
        pltpu.sync_copy(row_hbm.at[pl.ds(off, CK)], rowv0)
        pltpu.sync_copy(col_hbm.at[pl.ds(off, CK)], colv0)
        pltpu.async_copy(c_hbm.at[rowv0], cv0, semc)
        pltpu.sync_copy(stc_hbm.at[pl.ds(off, CK)], sv0)
        pltpu.make_async_copy(c_hbm.at[rowv0], cv0, semc).wait()
        _add_rows(cv0, sv0, CK)
        pltpu.sync_copy(cv0, acc_sh.at[colv0], add=True)
        return 0
    lax.fori_loop(0, NFULL, body, 0)

    plsc.subcore_barrier()

    @pl.when(sid == 0)
    def _():
        pltpu.sync_copy(acc_sh, acc_out.at[cid])


@functools.partial(
    pl.kernel,
    out_type=jax.ShapeDtypeStruct((NC, TBL, DN), jnp.float32),
    mesh=_SC_MESH,
    scratch_types=[
        pltpu.VMEM((CK,), jnp.int32), pltpu.VMEM((CK,), jnp.int32),
        pltpu.VMEM((CK, DN), jnp.float32), pltpu.VMEM((CK, DN), jnp.float32),
        pltpu.VMEM_SHARED((TBL, DN), jnp.float32),
        pltpu.SemaphoreType.DMA,
    ],
)
def _p3a(c_hbm, stc_hbm, row_hbm, col_hbm, acc_out, *rest):
    _p3a_body(c_hbm, stc_hbm, row_hbm, col_hbm, acc_out, *rest)


def _p3b_body(enew_hbm, row_hbm, racc_out, rowv0, ev0, ew, racc_sh):
    cid = lax.axis_index("c")
    sid = lax.axis_index("s")
    wid = sid * NC + cid
    base = pl.multiple_of(wid * EPW, 128)

    zero16 = jnp.zeros((16,), jnp.float32)

    def fill(i, _):
        for j in range(8):
            ew[i, pl.ds(j * 16, 16)] = zero16
        return 0
    lax.fori_loop(0, CK, fill, 0)
    _zero_shared(ew, racc_sh, sid)
    plsc.subcore_barrier()

    def body(c, _):
        off = pl.multiple_of(base + c * CK, 128)
        pltpu.sync_copy(row_hbm.at[pl.ds(off, CK)], rowv0)
        pltpu.sync_copy(enew_hbm.at[pl.ds(off, CK)], ev0)

        def cp(i, _):
            ew[i, pl.ds(0, 16)] = ev0[i, pl.ds(0, 16)]
            ew[i, pl.ds(16, 16)] = ev0[i, pl.ds(16, 16)]
            return 0
        lax.fori_loop(0, CK, cp, 0)
        pltpu.sync_copy(ew, racc_sh.at[rowv0], add=True)
        return 0
    lax.fori_loop(0, NFULL, body, 0)

    plsc.subcore_barrier()

    @pl.when(sid == 0)
    def _():
        pltpu.sync_copy(racc_sh, racc_out.at[cid])


@functools.partial(
    pl.kernel,
    out_type=jax.ShapeDtypeStruct((NC, TBL, DN), jnp.float32),
    mesh=_SC_MESH,
    scratch_types=[
        pltpu.VMEM((CK,), jnp.int32), pltpu.VMEM((CK, 2 * DE), jnp.float32),
        pltpu.VMEM((CK, DN), jnp.float32),
        pltpu.VMEM_SHARED((TBL, DN), jnp.float32),
    ],
)
def _p3b(enew_hbm, row_hbm, racc_out, *rest):
    _p3b_body(enew_hbm, row_hbm, racc_out, *rest)


def _p0_body(x_ref, oh_ref, w_ref, u_ref, wu_ref, b_ref, out_ref):
    uw = u_ref[...] @ wu_ref[...]
    t = oh_ref[...] @ uw
    pad = jnp.zeros((NB, 2 * DN), jnp.float32)
    out_ref[...] = (x_ref[...] @ w_ref[...] + b_ref[...]
                    + jnp.concatenate([t, pad], axis=1))


def _p0(x, oh, w, u, wu, b):
    return pl.pallas_call(
        _p0_body,
        grid=(NGRID,),
        in_specs=[
            pl.BlockSpec((NB, DN), lambda i: (i, 0)),
            pl.BlockSpec((NB, G), lambda i: (i, 0)),
            pl.BlockSpec((DN, 3 * DN), lambda i: (0, 0)),
            pl.BlockSpec((G, DG), lambda i: (0, 0)),
            pl.BlockSpec((DG, DN), lambda i: (0, 0)),
            pl.BlockSpec((1, 3 * DN), lambda i: (0, 0)),
        ],
        out_specs=pl.BlockSpec((NB, 3 * DN), lambda i: (i, 0)),
        out_shape=jax.ShapeDtypeStruct((N, 3 * DN), jnp.float32),
        compiler_params=pltpu.CompilerParams(
            dimension_semantics=("arbitrary",)),
    )(x, oh, w, u, wu, b)


def _p2_body(g_ref, ea_ref, wea_ref, w2_ref, w1b_ref, eb2_ref,
             enew_ref, enew32_ref, stc_ref):
    h = jnp.maximum(g_ref[...] + ea_ref[...] @ wea_ref[...], 0.0)
    enew = h @ w2_ref[...] + eb2_ref[...]
    enew_ref[...] = enew
    enew32_ref[...] = jnp.concatenate(
        [enew, jnp.ones((EBLK, DE), jnp.float32)], axis=1)
    m2 = w2_ref[...] @ w1b_ref[...]
    c2 = eb2_ref[...] @ w1b_ref[...]
    stc_ref[...] = h @ m2 + c2


def _p2(g, edge_attr, wea, ew2, w1b, eb2):
    return pl.pallas_call(
        _p2_body,
        grid=(EGRID,),
        in_specs=[
            pl.BlockSpec((EBLK, DN), lambda i: (i, 0)),
            pl.BlockSpec((EBLK, DE), lambda i: (i, 0)),
            pl.BlockSpec((DE, DN), lambda i: (0, 0)),
            pl.BlockSpec((DN, DE), lambda i: (0, 0)),
            pl.BlockSpec((DE, DN), lambda i: (0, 0)),
            pl.BlockSpec((1, DE), lambda i: (0, 0)),
        ],
        out_specs=[
            pl.BlockSpec((EBLK, DE), lambda i: (i, 0)),
            pl.BlockSpec((EBLK, 2 * DE), lambda i: (i, 0)),
            pl.BlockSpec((EBLK, DN), lambda i: (i, 0)),
        ],
        out_shape=[
            jax.ShapeDtypeStruct((E2, DE), jnp.float32),
            jax.ShapeDtypeStruct((E2, 2 * DE), jnp.float32),
            jax.ShapeDtypeStruct((E2, DN), jnp.float32),
        ],
        compiler_params=pltpu.CompilerParams(
            dimension_semantics=("arbitrary",)),
    )(g, edge_attr, wea, ew2, w1b, eb2)


def _p4_body(a0, a1, r0, r1, x_ref, oh_ref, u_ref,
             n1b1_, n1w2_, n1b2_, w2a, w2b, w2c, n2b1_, n2w2_, n2b2_,
             gw1_, gb1_, gw2_, gb2_,
             xnew_ref, nsum_ref, gacc_ref, unew_ref):
    i = pl.program_id(0)
    acct = a0[...] + a1[...]
    h1 = jnp.maximum(acct + n1b1_[...], 0.0) @ n1w2_[...] + n1b2_[...]
    uw2 = u_ref[...] @ w2c[...]
    pre2 = (x_ref[...] @ w2a[...] + h1 @ w2b[...]
            + oh_ref[...] @ uw2 + n2b1_[...])
    xn = jnp.maximum(pre2, 0.0) @ n2w2_[...] + n2b2_[...]
    xnew_ref[...] = xn
    oh = oh_ref[...]
    tdims = (((0,), (0,)), ((), ()))
    nsum_c = lax.dot_general(oh, xn, tdims,
                             preferred_element_type=jnp.float32)
    vals = jnp.concatenate(
        [(r0[...] + r1[...])[:, :2 * DE],
         jnp.ones((NB, DE), jnp.float32),
         jnp.zeros((NB, DN - 3 * DE), jnp.float32)], axis=1)
    gacc_c = lax.dot_general(oh, vals, tdims,
                             preferred_element_type=jnp.float32)

    @pl.when(i == 0)
    def _():
        nsum_ref[...] = nsum_c
        gacc_ref[...] = gacc_c

    @pl.when(i > 0)
    def _():
        nsum_ref[...] += nsum_c
        gacc_ref[...] += gacc_c

    @pl.when(i == NGRID - 1)
    def _():
        nsum = nsum_ref[...]
        gacc = gacc_ref[...]
        esum = gacc[:, :DE]
        ecnt = gacc[:, DE:DE + 1]
        ncnt = gacc[:, 2 * DE:2 * DE + 1]
        node_info = nsum / jnp.maximum(ncnt, 1.0)
        edge_info = esum / jnp.maximum(ecnt, 1.0)
        g_in = jnp.concatenate([u_ref[...], node_info, edge_info], axis=1)
        unew_ref[...] = (jnp.maximum(g_in @ gw1_[...] + gb1_[...], 0.0)
                         @ gw2_[...] + gb2_[...])


def _p4(acc0, acc1, racc0, racc1, x, oh, u,
        n1b1, n1w2, n1b2, w2a, w2b, w2c, n2b1, n2w2, n2b2,
        gw1, gb1, gw2, gb2):
    def full(shape):
        return pl.BlockSpec(shape, lambda i: tuple(0 for _ in shape))
    return pl.pallas_call(
        _p4_body,
        grid=(NGRID,),
        in_specs=[
            pl.BlockSpec((NB, DN), lambda i: (i, 0)),
            pl.BlockSpec((NB, DN), lambda i: (i, 0)),
            pl.BlockSpec((NB, DN), lambda i: (i, 0)),
            pl.BlockSpec((NB, DN), lambda i: (i, 0)),
            pl.BlockSpec((NB, DN), lambda i: (i, 0)),
            pl.BlockSpec((NB, G), lambda i: (i, 0)),
            full((G, DG)),
            full((1, DN)), full((DN, DN)), full((1, DN)),
            full((DN, DN)), full((DN, DN)), full((DG, DN)),
            full((1, DN)), full((DN, DN)), full((1, DN)),
            full((DG + DN + DE, DN)), full((1, DN)),
            full((DN, DG)), full((1, DG)),
        ],
        out_specs=[
            pl.BlockSpec((NB, DN), lambda i: (i, 0)),
            pl.BlockSpec((G, DN), lambda i: (0, 0)),
            pl.BlockSpec((G, DN), lambda i: (0, 0)),
            pl.BlockSpec((G, DG), lambda i: (0, 0)),
        ],
        out_shape=[
            jax.ShapeDtypeStruct((N, DN), jnp.float32),
            jax.ShapeDtypeStruct((G, DN), jnp.float32),
            jax.ShapeDtypeStruct((G, DN), jnp.float32),
            jax.ShapeDtypeStruct((G, DG), jnp.float32),
        ],
        compiler_params=pltpu.CompilerParams(
            dimension_semantics=("arbitrary",)),
    )(acc0, acc1, racc0, racc1, x, oh, u,
      n1b1, n1w2, n1b2, w2a, w2b, w2c, n2b1, n2w2, n2b2,
      gw1, gb1, gw2, gb2)


def kernel(x, edge_index, edge_attr, u, batch,
           ew1, eb1, ew2, eb2,
           n1w1, n1b1, n1w2, n1b2,
           n2w1, n2b1, n2w2, n2b2,
           gw1, gb1, gw2, gb2):
    pad_e = E2 - E
    row = jnp.concatenate([edge_index[0],
                           jnp.full((pad_e,), N, edge_index.dtype)])
    col = jnp.concatenate([edge_index[1],
                           jnp.full((pad_e,), N, edge_index.dtype)])
    ea_p = jnp.pad(edge_attr, ((0, pad_e), (0, 0)))
    oh = (batch[:, None] == jnp.arange(G, dtype=batch.dtype)[None, :])
    oh = oh.astype(jnp.float32)

    w0 = jnp.concatenate([ew1[:DN], ew1[DN:2 * DN], n1w1[:DN]], axis=1)
    b0 = jnp.concatenate([eb1, jnp.zeros((2 * DN,), jnp.float32)])[None, :]
    abc = jnp.pad(_p0(x, oh, w0, u, ew1[2 * DN + DE:], b0),
                  ((0, TBL - N), (0, 0)))
    a_t = abc[:, :DN]
    b_t = abc[:, DN:2 * DN]
    c_t = abc[:, 2 * DN:]

    g = _p1(a_t, b_t, row, col)

    e_new, e_new32, s_tc = _p2(g, ea_p, ew1[2 * DN:2 * DN + DE], ew2,
                               n1w1[DN:DN + DE], eb2[None, :])

    acc_p = _p3a(c_t, s_tc, row, col)
    racc_p = _p3b(e_new32, row)

    x_new, _, _, u_new = _p4(
        acc_p[0], acc_p[1], racc_p[0], racc_p[1],
        x, oh, u,
        n1b1[None, :], n1w2, n1b2[None, :],
        n2w1[:DN], n2w1[DN:DN + H], n2w1[DN + H:],
        n2b1[None, :], n2w2, n2b2[None, :],
        gw1, gb1[None, :], gw2, gb2[None, :])

    return (x_new, e_new[:E], u_new)


# speedup vs baseline: 1.1408x; 1.0181x over previous
import functools

import jax
import jax.numpy as jnp
from jax import lax
from jax.experimental import pallas as pl
from jax.experimental.pallas import tpu as pltpu, tpu_sc as plsc

N = 10000
E = 320000
G = 16
DN = 128
DE = 16
DG = 32
H = 128

NC = 2
NS = 16
NW = NC * NS
CK = 128
NFULL = 79
EPW = NFULL * CK
E2 = EPW * NW
TBL = 10176

NB = 200
NGRID = N // NB
EBLK = 512
EGRID = E2 // EBLK

_SC_MESH = plsc.VectorSubcoreMesh(
    core_axis_name="c", subcore_axis_name="s", num_cores=NC, num_subcores=NS)


def _add_rows(dst, src, nrows):
    def body(i, _):
        for j in range(8):
            sl = pl.ds(j * 16, 16)
            dst[i, sl] = dst[i, sl] + src[i, sl]
        return 0
    lax.fori_loop(0, nrows, body, 0)


def _p1_body(a_hbm, b_hbm, row_hbm, col_hbm, g_hbm,
             rowv, colv, av, bv, sem0, sem1):
    wid = lax.axis_index("s") * NC + lax.axis_index("c")
    base = pl.multiple_of(wid * EPW, 128)

    def body(c, _):
        off = pl.multiple_of(base + c * CK, 128)
        pltpu.sync_copy(row_hbm.at[pl.ds(off, CK)], rowv)
        pltpu.sync_copy(col_hbm.at[pl.ds(off, CK)], colv)
        d0 = pltpu.async_copy(a_hbm.at[rowv], av, sem0)
        d1 = pltpu.async_copy(b_hbm.at[colv], bv, sem1)
        d0.wait()
        d1.wait()
        _add_rows(av, bv, CK)
        pltpu.sync_copy(av, g_hbm.at[pl.ds(off, CK)])
        return 0
    lax.fori_loop(0, NFULL, body, 0)


@functools.partial(
    pl.kernel,
    out_type=jax.ShapeDtypeStruct((E2, DN), jnp.float32),
    mesh=_SC_MESH,
    scratch_types=[
        pltpu.VMEM((CK,), jnp.int32), pltpu.VMEM((CK,), jnp.int32),
        pltpu.VMEM((CK, DN), jnp.float32), pltpu.VMEM((CK, DN), jnp.float32),
        pltpu.SemaphoreType.DMA, pltpu.SemaphoreType.DMA,
    ],
)
def _p1(a_hbm, b_hbm, row_hbm, col_hbm, g_hbm, *rest):
    _p1_body(a_hbm, b_hbm, row_hbm, col_hbm, g_hbm, *rest)


def _zero_shared(zb, sh, sid):
    rps = TBL // NS
    zbase = sid * rps
    nfull = rps // CK

    def body(k, _):
        pltpu.sync_copy(zb, sh.at[pl.ds(zbase + k * CK, CK)])
        return 0
    lax.fori_loop(0, nfull, body, 0)
    rem = rps - nfull * CK
    if rem:
        pltpu.sync_copy(zb.at[pl.ds(0, rem)],
                        sh.at[pl.ds(zbase + nfull * CK, rem)])


def _p3a_body(c_hbm, stc_hbm, row_hbm, col_hbm, acc_out,
              rowv0, colv0, cv0, sv0, acc_sh, semc):
    cid = lax.axis_index("c")
    sid = lax.axis_index("s")
    wid = sid * NC + cid
    base = pl.multiple_of(wid * EPW, 128)

    zero16 = jnp.zeros((16,), jnp.float32)

    def fill(i, _):
        for j in range(8):
            cv0[i, pl.ds(j * 16, 16)] = zero16
        return 0
    lax.fori_loop(0, CK, fill, 0)
    _zero_shared(cv0, acc_sh, sid)
    plsc.subcore_barrier()

    def body(c, _):
        off = pl.multiple_of(base + c * CK, 128)
        pltpu.sync_copy(row_hbm.at[pl.ds(off, CK)], rowv0)
        pltpu.sync_copy(col_hbm.at[pl.ds(off, CK)], colv0)
        pltpu.async_copy(c_hbm.at[rowv0], cv0, semc)
        pltpu.sync_copy(stc_hbm.at[pl.ds(off, CK)], sv0)
        pltpu.make_async_copy(c_hbm.at[rowv0], cv0, semc).wait()
        _add_rows(cv0, sv0, CK)
        pltpu.sync_copy(cv0, acc_sh.at[colv0], add=True)
        return 0
    lax.fori_loop(0, NFULL, body, 0)

    plsc.subcore_barrier()

    @pl.when(sid == 0)
    def _():
        pltpu.sync_copy(acc_sh, acc_out.at[cid])


@functools.partial(
    pl.kernel,
    out_type=jax.ShapeDtypeStruct((NC, TBL, DN), jnp.float32),
    mesh=_SC_MESH,
    scratch_types=[
        pltpu.VMEM((CK,), jnp.int32), pltpu.VMEM((CK,), jnp.int32),
        pltpu.VMEM((CK, DN), jnp.float32), pltpu.VMEM((CK, DN), jnp.float32),
        pltpu.VMEM_SHARED((TBL, DN), jnp.float32),
        pltpu.SemaphoreType.DMA,
    ],
)
def _p3a(c_hbm, stc_hbm, row_hbm, col_hbm, acc_out, *rest):
    _p3a_body(c_hbm, stc_hbm, row_hbm, col_hbm, acc_out, *rest)


def _p3b_body(enew_hbm, row_hbm, racc_out, rowv0, ev0, ew, racc_sh):
    cid = lax.axis_index("c")
    sid = lax.axis_index("s")
    wid = sid * NC + cid
    base = pl.multiple_of(wid * EPW, 128)

    zero16 = jnp.zeros((16,), jnp.float32)

    def fill(i, _):
        for j in range(8):
            ew[i, pl.ds(j * 16, 16)] = zero16
        return 0
    lax.fori_loop(0, CK, fill, 0)
    _zero_shared(ew, racc_sh, sid)
    plsc.subcore_barrier()

    def body(c, _):
        off = pl.multiple_of(base + c * CK, 128)
        pltpu.sync_copy(row_hbm.at[pl.ds(off, CK)], rowv0)
        pltpu.sync_copy(enew_hbm.at[pl.ds(off, CK)], ev0)

        def cp(i, _):
            ew[i, pl.ds(0, 16)] = ev0[i, pl.ds(0, 16)]
            ew[i, pl.ds(16, 16)] = ev0[i, pl.ds(16, 16)]
            return 0
        lax.fori_loop(0, CK, cp, 0)
        pltpu.sync_copy(ew, racc_sh.at[rowv0], add=True)
        return 0
    lax.fori_loop(0, NFULL, body, 0)

    plsc.subcore_barrier()

    @pl.when(sid == 0)
    def _():
        pltpu.sync_copy(racc_sh, racc_out.at[cid])


@functools.partial(
    pl.kernel,
    out_type=jax.ShapeDtypeStruct((NC, TBL, DN), jnp.float32),
    mesh=_SC_MESH,
    scratch_types=[
        pltpu.VMEM((CK,), jnp.int32), pltpu.VMEM((CK, 2 * DE), jnp.float32),
        pltpu.VMEM((CK, DN), jnp.float32),
        pltpu.VMEM_SHARED((TBL, DN), jnp.float32),
    ],
)
def _p3b(enew_hbm, row_hbm, racc_out, *rest):
    _p3b_body(enew_hbm, row_hbm, racc_out, *rest)


def _p0_body(x_ref, oh_ref, w_ref, u_ref, wu_ref, b_ref, out_ref):
    uw = u_ref[...] @ wu_ref[...]
    t = oh_ref[...] @ uw
    pad = jnp.zeros((NB, 2 * DN), jnp.float32)
    out_ref[...] = (x_ref[...] @ w_ref[...] + b_ref[...]
                    + jnp.concatenate([t, pad], axis=1))


def _p0(x, oh, w, u, wu, b):
    return pl.pallas_call(
        _p0_body,
        grid=(NGRID,),
        in_specs=[
            pl.BlockSpec((NB, DN), lambda i: (i, 0)),
            pl.BlockSpec((NB, G), lambda i: (i, 0)),
            pl.BlockSpec((DN, 3 * DN), lambda i: (0, 0)),
            pl.BlockSpec((G, DG), lambda i: (0, 0)),
            pl.BlockSpec((DG, DN), lambda i: (0, 0)),
            pl.BlockSpec((1, 3 * DN), lambda i: (0, 0)),
        ],
        out_specs=pl.BlockSpec((NB, 3 * DN), lambda i: (i, 0)),
        out_shape=jax.ShapeDtypeStruct((N, 3 * DN), jnp.float32),
        compiler_params=pltpu.CompilerParams(
            dimension_semantics=("arbitrary",)),
    )(x, oh, w, u, wu, b)


def _p2_body(g_ref, ea_ref, wea_ref, w2_ref, w1b_ref, eb2_ref,
             enew_ref, enew32_ref, stc_ref):
    h = jnp.maximum(g_ref[...] + ea_ref[...] @ wea_ref[...], 0.0)
    enew = h @ w2_ref[...] + eb2_ref[...]
    enew_ref[...] = enew
    enew32_ref[...] = jnp.concatenate(
        [enew, jnp.ones((EBLK, DE), jnp.float32)], axis=1)
    m2 = w2_ref[...] @ w1b_ref[...]
    c2 = eb2_ref[...] @ w1b_ref[...]
    stc_ref[...] = h @ m2 + c2


def _p2(g, edge_attr, wea, ew2, w1b, eb2):
    return pl.pallas_call(
        _p2_body,
        grid=(EGRID,),
        in_specs=[
            pl.BlockSpec((EBLK, DN), lambda i: (i, 0)),
            pl.BlockSpec((EBLK, DE), lambda i: (i, 0)),
            pl.BlockSpec((DE, DN), lambda i: (0, 0)),
            pl.BlockSpec((DN, DE), lambda i: (0, 0)),
            pl.BlockSpec((DE, DN), lambda i: (0, 0)),
            pl.BlockSpec((1, DE), lambda i: (0, 0)),
        ],
        out_specs=[
            pl.BlockSpec((EBLK, DE), lambda i: (i, 0)),
            pl.BlockSpec((EBLK, 2 * DE), lambda i: (i, 0)),
            pl.BlockSpec((EBLK, DN), lambda i: (i, 0)),
        ],
        out_shape=[
            jax.ShapeDtypeStruct((E2, DE), jnp.float32),
            jax.ShapeDtypeStruct((E2, 2 * DE), jnp.float32),
            jax.ShapeDtypeStruct((E2, DN), jnp.float32),
        ],
        compiler_params=pltpu.CompilerParams(
            dimension_semantics=("arbitrary",)),
    )(g, edge_attr, wea, ew2, w1b, eb2)


def _p4_body(ap, rp, x_ref, oh_ref, u_ref,
             n1b1_, n1w2_, n1b2_, w2a, w2b, w2c, n2b1_, n2w2_, n2b2_,
             gw1_, gb1_, gw2_, gb2_,
             xnew_ref, nsum_ref, gacc_ref, unew_ref):
    i = pl.program_id(0)
    acct = ap[0] + ap[1]
    h1 = jnp.maximum(acct + n1b1_[...], 0.0) @ n1w2_[...] + n1b2_[...]
    uw2 = u_ref[...] @ w2c[...]
    pre2 = (x_ref[...] @ w2a[...] + h1 @ w2b[...]
            + oh_ref[...] @ uw2 + n2b1_[...])
    xn = jnp.maximum(pre2, 0.0) @ n2w2_[...] + n2b2_[...]
    xnew_ref[...] = xn
    oh = oh_ref[...]
    tdims = (((0,), (0,)), ((), ()))
    nsum_c = lax.dot_general(oh, xn, tdims,
                             preferred_element_type=jnp.float32)
    vals = jnp.concatenate(
        [(rp[0] + rp[1])[:, :2 * DE],
         jnp.ones((NB, DE), jnp.float32),
         jnp.zeros((NB, DN - 3 * DE), jnp.float32)], axis=1)
    gacc_c = lax.dot_general(oh, vals, tdims,
                             preferred_element_type=jnp.float32)

    @pl.when(i == 0)
    def _():
        nsum_ref[...] = nsum_c
        gacc_ref[...] = gacc_c

    @pl.when(i > 0)
    def _():
        nsum_ref[...] += nsum_c
        gacc_ref[...] += gacc_c

    @pl.when(i == NGRID - 1)
    def _():
        nsum = nsum_ref[...]
        gacc = gacc_ref[...]
        esum = gacc[:, :DE]
        ecnt = gacc[:, DE:DE + 1]
        ncnt = gacc[:, 2 * DE:2 * DE + 1]
        node_info = nsum / jnp.maximum(ncnt, 1.0)
        edge_info = esum / jnp.maximum(ecnt, 1.0)
        g_in = jnp.concatenate([u_ref[...], node_info, edge_info], axis=1)
        unew_ref[...] = (jnp.maximum(g_in @ gw1_[...] + gb1_[...], 0.0)
                         @ gw2_[...] + gb2_[...])


def _p4(acc_p, racc_p, x, oh, u,
        n1b1, n1w2, n1b2, w2a, w2b, w2c, n2b1, n2w2, n2b2,
        gw1, gb1, gw2, gb2):
    def full(shape):
        return pl.BlockSpec(shape, lambda i: tuple(0 for _ in shape))
    return pl.pallas_call(
        _p4_body,
        grid=(NGRID,),
        in_specs=[
            pl.BlockSpec((NC, NB, DN), lambda i: (0, i, 0)),
            pl.BlockSpec((NC, NB, DN), lambda i: (0, i, 0)),
            pl.BlockSpec((NB, DN), lambda i: (i, 0)),
            pl.BlockSpec((NB, G), lambda i: (i, 0)),
            full((G, DG)),
            full((1, DN)), full((DN, DN)), full((1, DN)),
            full((DN, DN)), full((DN, DN)), full((DG, DN)),
            full((1, DN)), full((DN, DN)), full((1, DN)),
            full((DG + DN + DE, DN)), full((1, DN)),
            full((DN, DG)), full((1, DG)),
        ],
        out_specs=[
            pl.BlockSpec((NB, DN), lambda i: (i, 0)),
            pl.BlockSpec((G, DN), lambda i: (0, 0)),
            pl.BlockSpec((G, DN), lambda i: (0, 0)),
            pl.BlockSpec((G, DG), lambda i: (0, 0)),
        ],
        out_shape=[
            jax.ShapeDtypeStruct((N, DN), jnp.float32),
            jax.ShapeDtypeStruct((G, DN), jnp.float32),
            jax.ShapeDtypeStruct((G, DN), jnp.float32),
            jax.ShapeDtypeStruct((G, DG), jnp.float32),
        ],
        compiler_params=pltpu.CompilerParams(
            dimension_semantics=("arbitrary",)),
    )(acc_p, racc_p, x, oh, u,
      n1b1, n1w2, n1b2, w2a, w2b, w2c, n2b1, n2w2, n2b2,
      gw1, gb1, gw2, gb2)


def kernel(x, edge_index, edge_attr, u, batch,
           ew1, eb1, ew2, eb2,
           n1w1, n1b1, n1w2, n1b2,
           n2w1, n2b1, n2w2, n2b2,
           gw1, gb1, gw2, gb2):
    pad_e = E2 - E
    row = jnp.concatenate([edge_index[0],
                           jnp.full((pad_e,), N, edge_index.dtype)])
    col = jnp.concatenate([edge_index[1],
                           jnp.full((pad_e,), N, edge_index.dtype)])
    ea_p = jnp.pad(edge_attr, ((0, pad_e), (0, 0)))
    oh = (batch[:, None] == jnp.arange(G, dtype=batch.dtype)[None, :])
    oh = oh.astype(jnp.float32)

    w0 = jnp.concatenate([ew1[:DN], ew1[DN:2 * DN], n1w1[:DN]], axis=1)
    b0 = jnp.concatenate([eb1, jnp.zeros((2 * DN,), jnp.float32)])[None, :]
    abc = jnp.pad(_p0(x, oh, w0, u, ew1[2 * DN + DE:], b0),
                  ((0, TBL - N), (0, 0)))
    a_t = abc[:, :DN]
    b_t = abc[:, DN:2 * DN]
    c_t = abc[:, 2 * DN:]

    g = _p1(a_t, b_t, row, col)

    e_new, e_new32, s_tc = _p2(g, ea_p, ew1[2 * DN:2 * DN + DE], ew2,
                               n1w1[DN:DN + DE], eb2[None, :])

    acc_p = _p3a(c_t, s_tc, row, col)
    racc_p = _p3b(e_new32, row)

    x_new, _, _, u_new = _p4(
        acc_p, racc_p,
        x, oh, u,
        n1b1[None, :], n1w2, n1b2[None, :],
        n2w1[:DN], n2w1[DN:DN + H], n2w1[DN + H:],
        n2b1[None, :], n2w2, n2b2[None, :],
        gw1, gb1[None, :], gw2, gb2[None, :])

    return (x_new, e_new[:E], u_new)


# speedup vs baseline: 1.1607x; 1.0175x over previous
import functools

import jax
import jax.numpy as jnp
from jax import lax
from jax.experimental import pallas as pl
from jax.experimental.pallas import tpu as pltpu, tpu_sc as plsc

N = 10000
E = 320000
G = 16
DN = 128
DE = 16
DG = 32
H = 128

NC = 2
NS = 16
NW = NC * NS
CK = 128
NFULL = 79
EPW = NFULL * CK
E2 = EPW * NW
TBL = 10176

NB = 200
NGRID = N // NB
EBLK = 512
EGRID = E2 // EBLK

_SC_MESH = plsc.VectorSubcoreMesh(
    core_axis_name="c", subcore_axis_name="s", num_cores=NC, num_subcores=NS)


def _add_rows(dst, src, nrows):
    def body(i, _):
        for j in range(8):
            sl = pl.ds(j * 16, 16)
            dst[i, sl] = dst[i, sl] + src[i, sl]
        return 0
    lax.fori_loop(0, nrows, body, 0)


def _p1_body(a_hbm, b_hbm, row_hbm, col_hbm, g_hbm,
             rowv, colv, av, bv, sem0, sem1):
    wid = lax.axis_index("s") * NC + lax.axis_index("c")
    base = pl.multiple_of(wid * EPW, 128)

    def body(c, _):
        off = pl.multiple_of(base + c * CK, 128)
        pltpu.sync_copy(row_hbm.at[pl.ds(off, CK)], rowv)
        pltpu.sync_copy(col_hbm.at[pl.ds(off, CK)], colv)
        d0 = pltpu.async_copy(a_hbm.at[rowv], av, sem0)
        d1 = pltpu.async_copy(b_hbm.at[colv], bv, sem1)
        d0.wait()
        d1.wait()
        _add_rows(av, bv, CK)
        pltpu.sync_copy(av, g_hbm.at[pl.ds(off, CK)])
        return 0
    lax.fori_loop(0, NFULL, body, 0)


@functools.partial(
    pl.kernel,
    out_type=jax.ShapeDtypeStruct((E2, DN), jnp.float32),
    mesh=_SC_MESH,
    scratch_types=[
        pltpu.VMEM((CK,), jnp.int32), pltpu.VMEM((CK,), jnp.int32),
        pltpu.VMEM((CK, DN), jnp.float32), pltpu.VMEM((CK, DN), jnp.float32),
        pltpu.SemaphoreType.DMA, pltpu.SemaphoreType.DMA,
    ],
)
def _p1(a_hbm, b_hbm, row_hbm, col_hbm, g_hbm, *rest):
    _p1_body(a_hbm, b_hbm, row_hbm, col_hbm, g_hbm, *rest)


def _zero_shared(zb, sh, sid):
    rps = TBL // NS
    zbase = sid * rps
    nfull = rps // CK

    def body(k, _):
        pltpu.sync_copy(zb, sh.at[pl.ds(zbase + k * CK, CK)])
        return 0
    lax.fori_loop(0, nfull, body, 0)
    rem = rps - nfull * CK
    if rem:
        pltpu.sync_copy(zb.at[pl.ds(0, rem)],
                        sh.at[pl.ds(zbase + nfull * CK, rem)])


def _p3a_body(c_hbm, stc_hbm, row_hbm, col_hbm, acc_out,
              rowv0, colv0, cv0, sv0, acc_sh, semc):
    cid = lax.axis_index("c")
    sid = lax.axis_index("s")
    wid = sid * NC + cid
    base = pl.multiple_of(wid * EPW, 128)

    zero16 = jnp.zeros((16,), jnp.float32)

    def fill(i, _):
        for j in range(8):
            cv0[i, pl.ds(j * 16, 16)] = zero16
        return 0
    lax.fori_loop(0, CK, fill, 0)
    _zero_shared(cv0, acc_sh, sid)
    plsc.subcore_barrier()

    def body(c, _):
        off = pl.multiple_of(base + c * CK, 128)
        pltpu.sync_copy(row_hbm.at[pl.ds(off, CK)], rowv0)
        pltpu.sync_copy(col_hbm.at[pl.ds(off, CK)], colv0)
        pltpu.async_copy(c_hbm.at[rowv0], cv0, semc)
        pltpu.sync_copy(stc_hbm.at[pl.ds(off, CK)], sv0)
        pltpu.make_async_copy(c_hbm.at[rowv0], cv0, semc).wait()
        _add_rows(cv0, sv0, CK)
        pltpu.sync_copy(cv0, acc_sh.at[colv0], add=True)
        return 0
    lax.fori_loop(0, NFULL, body, 0)

    plsc.subcore_barrier()

    @pl.when(sid == 0)
    def _():
        pltpu.sync_copy(acc_sh, acc_out.at[cid])


@functools.partial(
    pl.kernel,
    out_type=jax.ShapeDtypeStruct((NC, TBL, DN), jnp.float32),
    mesh=_SC_MESH,
    scratch_types=[
        pltpu.VMEM((CK,), jnp.int32), pltpu.VMEM((CK,), jnp.int32),
        pltpu.VMEM((CK, DN), jnp.float32), pltpu.VMEM((CK, DN), jnp.float32),
        pltpu.VMEM_SHARED((TBL, DN), jnp.float32),
        pltpu.SemaphoreType.DMA,
    ],
)
def _p3a(c_hbm, stc_hbm, row_hbm, col_hbm, acc_out, *rest):
    _p3a_body(c_hbm, stc_hbm, row_hbm, col_hbm, acc_out, *rest)


def _p3b_body(enew_hbm, row_hbm, racc_out, rowv0, ev0, ew, racc_sh):
    cid = lax.axis_index("c")
    sid = lax.axis_index("s")
    wid = sid * NC + cid
    base = pl.multiple_of(wid * EPW, 128)

    zero16 = jnp.zeros((16,), jnp.float32)

    def fill(i, _):
        for j in range(8):
            ew[i, pl.ds(j * 16, 16)] = zero16
        return 0
    lax.fori_loop(0, CK, fill, 0)
    _zero_shared(ew, racc_sh, sid)
    plsc.subcore_barrier()

    def body(c, _):
        off = pl.multiple_of(base + c * CK, 128)
        pltpu.sync_copy(row_hbm.at[pl.ds(off, CK)], rowv0)
        pltpu.sync_copy(enew_hbm.at[pl.ds(off, CK)], ev0)

        def cp(i, _):
            ew[i, pl.ds(0, 16)] = ev0[i, pl.ds(0, 16)]
            ew[i, pl.ds(16, 16)] = ev0[i, pl.ds(16, 16)]
            return 0
        lax.fori_loop(0, CK, cp, 0)
        pltpu.sync_copy(ew, racc_sh.at[rowv0], add=True)
        return 0
    lax.fori_loop(0, NFULL, body, 0)

    plsc.subcore_barrier()

    @pl.when(sid == 0)
    def _():
        pltpu.sync_copy(racc_sh, racc_out.at[cid])


@functools.partial(
    pl.kernel,
    out_type=jax.ShapeDtypeStruct((NC, TBL, DN), jnp.float32),
    mesh=_SC_MESH,
    scratch_types=[
        pltpu.VMEM((CK,), jnp.int32), pltpu.VMEM((CK, 2 * DE), jnp.float32),
        pltpu.VMEM((CK, DN), jnp.float32),
        pltpu.VMEM_SHARED((TBL, DN), jnp.float32),
    ],
)
def _p3b(enew_hbm, row_hbm, racc_out, *rest):
    _p3b_body(enew_hbm, row_hbm, racc_out, *rest)


def _p0_body(x_ref, oh_ref, w_ref, u_ref, wu_ref, b_ref, out_ref):
    uw = u_ref[...] @ wu_ref[...]
    t = oh_ref[...] @ uw
    pad = jnp.zeros((NB, 2 * DN), jnp.float32)
    out_ref[...] = (x_ref[...] @ w_ref[...] + b_ref[...]
                    + jnp.concatenate([t, pad], axis=1))


def _p0(x, oh, w, u, wu, b):
    return pl.pallas_call(
        _p0_body,
        grid=(NGRID,),
        in_specs=[
            pl.BlockSpec((NB, DN), lambda i: (i, 0)),
            pl.BlockSpec((NB, G), lambda i: (i, 0)),
            pl.BlockSpec((DN, 3 * DN), lambda i: (0, 0)),
            pl.BlockSpec((G, DG), lambda i: (0, 0)),
            pl.BlockSpec((DG, DN), lambda i: (0, 0)),
            pl.BlockSpec((1, 3 * DN), lambda i: (0, 0)),
        ],
        out_specs=pl.BlockSpec((NB, 3 * DN), lambda i: (i, 0)),
        out_shape=jax.ShapeDtypeStruct((N, 3 * DN), jnp.float32),
        compiler_params=pltpu.CompilerParams(
            dimension_semantics=("arbitrary",)),
    )(x, oh, w, u, wu, b)


def _p2_body(g_ref, ea_ref, wea_ref, w2_ref, w1b_ref, eb2_ref,
             enew_ref, enew32_ref, stc_ref):
    h = jnp.maximum(g_ref[...] + ea_ref[...] @ wea_ref[...], 0.0)
    enew = h @ w2_ref[...] + eb2_ref[...]

    @pl.when(pl.program_id(0) < E // EBLK)
    def _():
        enew_ref[...] = enew
    enew32_ref[...] = jnp.concatenate(
        [enew, jnp.ones((EBLK, DE), jnp.float32)], axis=1)
    m2 = w2_ref[...] @ w1b_ref[...]
    c2 = eb2_ref[...] @ w1b_ref[...]
    stc_ref[...] = h @ m2 + c2


def _p2(g, edge_attr, wea, ew2, w1b, eb2):
    return pl.pallas_call(
        _p2_body,
        grid=(EGRID,),
        in_specs=[
            pl.BlockSpec((EBLK, DN), lambda i: (i, 0)),
            pl.BlockSpec((EBLK, DE),
                         lambda i: (jnp.minimum(i, E // EBLK - 1), 0)),
            pl.BlockSpec((DE, DN), lambda i: (0, 0)),
            pl.BlockSpec((DN, DE), lambda i: (0, 0)),
            pl.BlockSpec((DE, DN), lambda i: (0, 0)),
            pl.BlockSpec((1, DE), lambda i: (0, 0)),
        ],
        out_specs=[
            pl.BlockSpec((EBLK, DE),
                         lambda i: (jnp.minimum(i, E // EBLK - 1), 0)),
            pl.BlockSpec((EBLK, 2 * DE), lambda i: (i, 0)),
            pl.BlockSpec((EBLK, DN), lambda i: (i, 0)),
        ],
        out_shape=[
            jax.ShapeDtypeStruct((E, DE), jnp.float32),
            jax.ShapeDtypeStruct((E2, 2 * DE), jnp.float32),
            jax.ShapeDtypeStruct((E2, DN), jnp.float32),
        ],
        compiler_params=pltpu.CompilerParams(
            dimension_semantics=("arbitrary",)),
    )(g, edge_attr, wea, ew2, w1b, eb2)


def _p4_body(ap, rp, x_ref, oh_ref, u_ref,
             n1b1_, n1w2_, n1b2_, w2a, w2b, w2c, n2b1_, n2w2_, n2b2_,
             gw1_, gb1_, gw2_, gb2_,
             xnew_ref, nsum_ref, gacc_ref, unew_ref):
    i = pl.program_id(0)
    acct = ap[0] + ap[1]
    h1 = jnp.maximum(acct + n1b1_[...], 0.0) @ n1w2_[...] + n1b2_[...]
    uw2 = u_ref[...] @ w2c[...]
    pre2 = (x_ref[...] @ w2a[...] + h1 @ w2b[...]
            + oh_ref[...] @ uw2 + n2b1_[...])
    xn = jnp.maximum(pre2, 0.0) @ n2w2_[...] + n2b2_[...]
    xnew_ref[...] = xn
    oh = oh_ref[...]
    tdims = (((0,), (0,)), ((), ()))
    nsum_c = lax.dot_general(oh, xn, tdims,
                             preferred_element_type=jnp.float32)
    vals = jnp.concatenate(
        [(rp[0] + rp[1])[:, :2 * DE],
         jnp.ones((NB, DE), jnp.float32),
         jnp.zeros((NB, DN - 3 * DE), jnp.float32)], axis=1)
    gacc_c = lax.dot_general(oh, vals, tdims,
                             preferred_element_type=jnp.float32)

    @pl.when(i == 0)
    def _():
        nsum_ref[...] = nsum_c
        gacc_ref[...] = gacc_c

    @pl.when(i > 0)
    def _():
        nsum_ref[...] += nsum_c
        gacc_ref[...] += gacc_c

    @pl.when(i == NGRID - 1)
    def _():
        nsum = nsum_ref[...]
        gacc = gacc_ref[...]
        esum = gacc[:, :DE]
        ecnt = gacc[:, DE:DE + 1]
        ncnt = gacc[:, 2 * DE:2 * DE + 1]
        node_info = nsum / jnp.maximum(ncnt, 1.0)
        edge_info = esum / jnp.maximum(ecnt, 1.0)
        g_in = jnp.concatenate([u_ref[...], node_info, edge_info], axis=1)
        unew_ref[...] = (jnp.maximum(g_in @ gw1_[...] + gb1_[...], 0.0)
                         @ gw2_[...] + gb2_[...])


def _p4(acc_p, racc_p, x, oh, u,
        n1b1, n1w2, n1b2, w2a, w2b, w2c, n2b1, n2w2, n2b2,
        gw1, gb1, gw2, gb2):
    def full(shape):
        return pl.BlockSpec(shape, lambda i: tuple(0 for _ in shape))
    return pl.pallas_call(
        _p4_body,
        grid=(NGRID,),
        in_specs=[
            pl.BlockSpec((NC, NB, DN), lambda i: (0, i, 0)),
            pl.BlockSpec((NC, NB, DN), lambda i: (0, i, 0)),
            pl.BlockSpec((NB, DN), lambda i: (i, 0)),
            pl.BlockSpec((NB, G), lambda i: (i, 0)),
            full((G, DG)),
            full((1, DN)), full((DN, DN)), full((1, DN)),
            full((DN, DN)), full((DN, DN)), full((DG, DN)),
            full((1, DN)), full((DN, DN)), full((1, DN)),
            full((DG + DN + DE, DN)), full((1, DN)),
            full((DN, DG)), full((1, DG)),
        ],
        out_specs=[
            pl.BlockSpec((NB, DN), lambda i: (i, 0)),
            pl.BlockSpec((G, DN), lambda i: (0, 0)),
            pl.BlockSpec((G, DN), lambda i: (0, 0)),
            pl.BlockSpec((G, DG), lambda i: (0, 0)),
        ],
        out_shape=[
            jax.ShapeDtypeStruct((N, DN), jnp.float32),
            jax.ShapeDtypeStruct((G, DN), jnp.float32),
            jax.ShapeDtypeStruct((G, DN), jnp.float32),
            jax.ShapeDtypeStruct((G, DG), jnp.float32),
        ],
        compiler_params=pltpu.CompilerParams(
            dimension_semantics=("arbitrary",)),
    )(acc_p, racc_p, x, oh, u,
      n1b1, n1w2, n1b2, w2a, w2b, w2c, n2b1, n2w2, n2b2,
      gw1, gb1, gw2, gb2)


def kernel(x, edge_index, edge_attr, u, batch,
           ew1, eb1, ew2, eb2,
           n1w1, n1b1, n1w2, n1b2,
           n2w1, n2b1, n2w2, n2b2,
           gw1, gb1, gw2, gb2):
    pad_e = E2 - E
    row = jnp.concatenate([edge_index[0],
                           jnp.full((pad_e,), N, edge_index.dtype)])
    col = jnp.concatenate([edge_index[1],
                           jnp.full((pad_e,), N, edge_index.dtype)])
    oh = (batch[:, None] == jnp.arange(G, dtype=batch.dtype)[None, :])
    oh = oh.astype(jnp.float32)

    w0 = jnp.concatenate([ew1[:DN], ew1[DN:2 * DN], n1w1[:DN]], axis=1)
    b0 = jnp.concatenate([eb1, jnp.zeros((2 * DN,), jnp.float32)])[None, :]
    abc = jnp.pad(_p0(x, oh, w0, u, ew1[2 * DN + DE:], b0),
                  ((0, TBL - N), (0, 0)))
    a_t = abc[:, :DN]
    b_t = abc[:, DN:2 * DN]
    c_t = abc[:, 2 * DN:]

    g = _p1(a_t, b_t, row, col)

    e_new, e_new32, s_tc = _p2(g, edge_attr, ew1[2 * DN:2 * DN + DE], ew2,
                               n1w1[DN:DN + DE], eb2[None, :])

    acc_p = _p3a(c_t, s_tc, row, col)
    racc_p = _p3b(e_new32, row)

    x_new, _, _, u_new = _p4(
        acc_p, racc_p,
        x, oh, u,
        n1b1[None, :], n1w2, n1b2[None, :],
        n2w1[:DN], n2w1[DN:DN + H], n2w1[DN + H:],
        n2b1[None, :], n2w2, n2b2[None, :],
        gw1, gb1[None, :], gw2, gb2[None, :])

    return (x_new, e_new, u_new)
